# baseline (speedup 1.0000x reference)
import functools

import jax
import jax.numpy as jnp
import numpy as np
from jax import lax
from jax.experimental import pallas as pl
from jax.experimental.pallas import tpu as pltpu
from jax.experimental.pallas import tpu_sc as plsc

N_NODES = 10000
D = 128
CHUNK = 128
NC, NS = 2, 16
NW = NC * NS

_PERM = np.empty(D, dtype=np.int32)
for _g in range(D // 32):
    for _i in range(16):
        _PERM[_g * 32 + 2 * _i] = _g * 32 + _i
        _PERM[_g * 32 + 2 * _i + 1] = _g * 32 + 16 + _i


def _mm_body(x_ref, w_ref, o_ref):
    o_ref[...] = jnp.dot(x_ref[...], w_ref[...],
                         preferred_element_type=jnp.float32
                         ).astype(jnp.bfloat16)


def _matmul_bf16(x, w):
    m_blk = 1000
    grid = (N_NODES // m_blk,)
    return pl.pallas_call(
        _mm_body,
        grid=grid,
        in_specs=[
            pl.BlockSpec((m_blk, D), lambda i: (i, 0)),
            pl.BlockSpec((D, D), lambda i: (0, 0)),
        ],
        out_specs=pl.BlockSpec((m_blk, D), lambda i: (i, 0)),
        out_shape=jax.ShapeDtypeStruct((N_NODES, D), jnp.bfloat16),
    )(x, w)


def _comb_body(p_ref, b_ref, o_ref):
    o_ref[...] = p_ref[0] + p_ref[1] + b_ref[...]


def _combine(partials, b2d):
    m_blk = 1000
    grid = (N_NODES // m_blk,)
    return pl.pallas_call(
        _comb_body,
        grid=grid,
        in_specs=[
            pl.BlockSpec((2, m_blk, D), lambda i: (0, i, 0)),
            pl.BlockSpec((1, D), lambda i: (0, 0)),
        ],
        out_specs=pl.BlockSpec((m_blk, D), lambda i: (i, 0)),
        out_shape=jax.ShapeDtypeStruct((N_NODES, D), jnp.float32),
    )(partials, b2d)


def _make_sc_aggregate(nct):
    wb = 40
    n_wb = N_NODES // wb
    half = nct // 2
    mesh = plsc.VectorSubcoreMesh(core_axis_name="c", subcore_axis_name="s")

    @functools.partial(
        pl.kernel,
        mesh=mesh,
        compiler_params=pltpu.CompilerParams(use_tc_tiling_on_sc=False),
        out_type=jax.ShapeDtypeStruct((NC, N_NODES, D), jnp.float32),
        scratch_types=[
            pltpu.VMEM((half, CHUNK), jnp.int32),
            pltpu.VMEM((2 * half, CHUNK // 2), jnp.int32),
            pltpu.VMEM((half, CHUNK), jnp.float32),
            pltpu.VMEM((CHUNK, D // 2), jnp.int32),
            pltpu.VMEM((CHUNK, D // 2), jnp.int32),
            pltpu.VMEM((CHUNK // 2, D), jnp.float32),
            pltpu.VMEM((CHUNK // 2, D), jnp.float32),
            pltpu.VMEM_SHARED((N_NODES, D), jnp.float32),
            pltpu.SemaphoreType.DMA,
            pltpu.SemaphoreType.DMA,
            pltpu.SemaphoreType.DMA,
            pltpu.SemaphoreType.DMA,
        ],
    )
    def agg(src_hbm, dst_hbm, val_hbm, sup_hbm, out_hbm,
            src_a, dst_a, val_a, bbuf0, bbuf1, fbuf0, fbuf1, acc_sh,
            gsem0, gsem1, fsem0, fsem1):
        cid = lax.axis_index("c")
        sid = lax.axis_index("s")
        wid = cid * NS + sid
        bbufs = (bbuf0, bbuf1)
        gsems = (gsem0, gsem1)
        fbufs = (fbuf0, fbuf1)
        fsems = (fsem0, fsem1)

        def _zero_row(r, carry):
            for dd in range(D // 16):
                fbuf0[r, pl.ds(dd * 16, 16)] = jnp.zeros((16,), jnp.float32)
            return carry
        lax.fori_loop(0, wb, _zero_row, 0)
        for z in range((n_wb + NS - 1) // NS):
            zc = z * NS + sid

            @pl.when(zc < n_wb)
            def _():
                pltpu.sync_copy(fbuf0.at[pl.ds(0, wb)],
                                acc_sh.at[pl.ds(zc * wb, wb)])
        plsc.subcore_barrier()

        for h in range(2):
            pltpu.sync_copy(src_hbm.at[wid, pl.ds(h * half, half)], src_a)
            pltpu.sync_copy(dst_hbm.at[wid, pl.ds(h * 2 * half, 2 * half)],
                            dst_a)
            pltpu.sync_copy(val_hbm.at[wid, pl.ds(h * half, half)], val_a)
            pltpu.async_copy(sup_hbm.at[src_a.at[0]], bbuf0, gsem0)
            pltpu.async_copy(sup_hbm.at[src_a.at[1]], bbuf1, gsem1)

            def _pair(i, carry):
                for b in range(2):
                    c = 2 * i + b
                    o = 1 - b
                    buf, gsem = bbufs[b], gsems[b]

                    @pl.when(jnp.logical_and(c >= 1, c + 1 < half))
                    def _():
                        pltpu.async_copy(sup_hbm.at[src_a.at[c + 1]],
                                         bbufs[o], gsems[o])

                    pltpu.make_async_copy(sup_hbm.at[src_a.at[c]], buf,
                                          gsem).wait()

                    for q in range(2):
                        fbuf, fsem = fbufs[q], fsems[q]

                        @pl.when(c >= 1)
                        def _():
                            pltpu.make_async_copy(
                                fbuf, acc_sh.at[dst_a.at[0]], fsem).wait()

                        def _scale(g, cc):
                            vv = val_a[c, pl.ds(q * 64 + g * 16, 16)]
                            for l in range(16):
                                v = vv[l]
                                e = q * 64 + g * 16 + l
                                eo = g * 16 + l
                                for dd in range(D // 32):
                                    w = buf[e, pl.ds(dd * 16, 16)]
                                    lo = lax.bitcast_convert_type(
                                        w << 16, jnp.float32)
                                    hi = lax.bitcast_convert_type(
                                        w & jnp.int32(-65536), jnp.float32)
                                    fbuf[eo, pl.ds(dd * 32, 16)] = lo * v
                                    fbuf[eo, pl.ds(dd * 32 + 16, 16)] = hi * v
                            return cc
                        lax.fori_loop(0, 4, _scale, 0)
                        pltpu.async_copy(fbuf,
                                         acc_sh.at[dst_a.at[2 * c + q]],
                                         fsem, add=True)
                return carry
            lax.fori_loop(0, half // 2, _pair, 0)
            pltpu.make_async_copy(fbuf0, acc_sh.at[dst_a.at[0]], fsem0).wait()
            pltpu.make_async_copy(fbuf1, acc_sh.at[dst_a.at[0]], fsem1).wait()
        plsc.subcore_barrier()

        for z in range((n_wb + NS - 1) // NS):
            zc = z * NS + sid

            @pl.when(zc < n_wb)
            def _():
                pltpu.sync_copy(acc_sh.at[pl.ds(zc * wb, wb)],
                                out_hbm.at[cid, pl.ds(zc * wb, wb)])

    return agg


def kernel(edge_index, adjacency_values, input_feature, W, b):
    e = edge_index.shape[1]
    grain = NW * CHUNK * 16
    e_pad = ((e + grain - 1) // grain) * grain
    nct = e_pad // (NW * CHUNK)
    pad = e_pad - e
    src = jnp.pad(edge_index[0].astype(jnp.int32), (0, pad))
    dst = jnp.pad(edge_index[1].astype(jnp.int32), (0, pad))
    vals = jnp.pad(adjacency_values, (0, pad))
    src3 = src.reshape(NW, nct, CHUNK)
    dst3 = dst.reshape(NW, 2 * nct, CHUNK // 2)
    val3 = vals.reshape(NW, nct, CHUNK)

    support = _matmul_bf16(input_feature, W[:, _PERM])
    sup_i32 = jax.lax.bitcast_convert_type(
        support.reshape(N_NODES, D // 2, 2), jnp.int32)
    partials = _make_sc_aggregate(nct)(src3, dst3, val3, sup_i32)
    return _combine(partials, b.reshape(1, D))


# speedup vs baseline: 1.0831x; 1.0831x over previous
import functools

import jax
import jax.numpy as jnp
import numpy as np
from jax import lax
from jax.experimental import pallas as pl
from jax.experimental.pallas import tpu as pltpu
from jax.experimental.pallas import tpu_sc as plsc

N_NODES = 10000
D = 128
CHUNK = 128
NC, NS = 2, 16
NW = NC * NS

_PERM = np.empty(D, dtype=np.int32)
for _g in range(D // 32):
    for _i in range(16):
        _PERM[_g * 32 + 2 * _i] = _g * 32 + _i
        _PERM[_g * 32 + 2 * _i + 1] = _g * 32 + 16 + _i


def _mm_body(x_ref, w_ref, o_ref):
    o_ref[...] = jnp.dot(x_ref[...], w_ref[...],
                         preferred_element_type=jnp.float32
                         ).astype(jnp.bfloat16)


def _matmul_bf16(x, w):
    m_blk = 1000
    grid = (N_NODES // m_blk,)
    return pl.pallas_call(
        _mm_body,
        grid=grid,
        in_specs=[
            pl.BlockSpec((m_blk, D), lambda i: (i, 0)),
            pl.BlockSpec((D, D), lambda i: (0, 0)),
        ],
        out_specs=pl.BlockSpec((m_blk, D), lambda i: (i, 0)),
        out_shape=jax.ShapeDtypeStruct((N_NODES, D), jnp.bfloat16),
    )(x, w)


def _comb_body(p_ref, b_ref, o_ref):
    o_ref[...] = p_ref[0] + p_ref[1] + b_ref[...]


def _combine(partials, b2d):
    m_blk = 1000
    grid = (N_NODES // m_blk,)
    return pl.pallas_call(
        _comb_body,
        grid=grid,
        in_specs=[
            pl.BlockSpec((2, m_blk, D), lambda i: (0, i, 0)),
            pl.BlockSpec((1, D), lambda i: (0, 0)),
        ],
        out_specs=pl.BlockSpec((m_blk, D), lambda i: (i, 0)),
        out_shape=jax.ShapeDtypeStruct((N_NODES, D), jnp.float32),
    )(partials, b2d)


def _make_sc_aggregate(nct):
    wb = 40
    n_wb = N_NODES // wb
    half = nct // 2
    mesh = plsc.VectorSubcoreMesh(core_axis_name="c", subcore_axis_name="s")

    @functools.partial(
        pl.kernel,
        mesh=mesh,
        compiler_params=pltpu.CompilerParams(use_tc_tiling_on_sc=False),
        out_type=jax.ShapeDtypeStruct((NC, N_NODES, D), jnp.float32),
        scratch_types=[
            pltpu.VMEM((half, CHUNK), jnp.int32),
            pltpu.VMEM((2 * half, CHUNK // 2), jnp.int32),
            pltpu.VMEM((half, CHUNK), jnp.float32),
            pltpu.VMEM((CHUNK, D // 2), jnp.int32),
            pltpu.VMEM((CHUNK, D // 2), jnp.int32),
            pltpu.VMEM((CHUNK // 2, D), jnp.float32),
            pltpu.VMEM((CHUNK // 2, D), jnp.float32),
            pltpu.VMEM_SHARED((N_NODES, D), jnp.float32),
            pltpu.SemaphoreType.DMA,
            pltpu.SemaphoreType.DMA,
            pltpu.SemaphoreType.DMA,
            pltpu.SemaphoreType.DMA,
        ],
    )
    def agg(src_hbm, dst_hbm, val_hbm, sup_hbm, out_hbm,
            src_a, dst_a, val_a, bbuf0, bbuf1, fbuf0, fbuf1, acc_sh,
            gsem0, gsem1, fsem0, fsem1):
        cid = lax.axis_index("c")
        sid = lax.axis_index("s")
        wid = cid * NS + sid
        bbufs = (bbuf0, bbuf1)
        gsems = (gsem0, gsem1)
        fbufs = (fbuf0, fbuf1)
        fsems = (fsem0, fsem1)

        def _zero_row(r, carry):
            for dd in range(D // 16):
                fbuf0[r, pl.ds(dd * 16, 16)] = jnp.zeros((16,), jnp.float32)
            return carry
        lax.fori_loop(0, wb, _zero_row, 0)
        for z in range((n_wb + NS - 1) // NS):
            zc = z * NS + sid

            @pl.when(zc < n_wb)
            def _():
                pltpu.sync_copy(fbuf0.at[pl.ds(0, wb)],
                                acc_sh.at[pl.ds(zc * wb, wb)])
        plsc.subcore_barrier()

        for h in range(2):
            pltpu.sync_copy(src_hbm.at[wid, pl.ds(h * half, half)], src_a)
            pltpu.sync_copy(dst_hbm.at[wid, pl.ds(h * 2 * half, 2 * half)],
                            dst_a)
            pltpu.sync_copy(val_hbm.at[wid, pl.ds(h * half, half)], val_a)
            pltpu.async_copy(sup_hbm.at[src_a.at[0]], bbuf0, gsem0)
            pltpu.async_copy(sup_hbm.at[src_a.at[1]], bbuf1, gsem1)

            def _pair(i, carry):
                for b in range(2):
                    c = 2 * i + b
                    o = 1 - b
                    buf, gsem = bbufs[b], gsems[b]

                    @pl.when(jnp.logical_and(c >= 1, c + 1 < half))
                    def _():
                        pltpu.async_copy(sup_hbm.at[src_a.at[c + 1]],
                                         bbufs[o], gsems[o])

                    pltpu.make_async_copy(sup_hbm.at[src_a.at[c]], buf,
                                          gsem).wait()

                    for q in range(2):
                        fbuf, fsem = fbufs[q], fsems[q]

                        @pl.when(c >= 1)
                        def _():
                            pltpu.make_async_copy(
                                fbuf, acc_sh.at[dst_a.at[0]], fsem).wait()

                        @plsc.parallel_loop(0, 4, unroll=2)
                        def _scale(g):
                            vv = val_a[c, pl.ds(q * 64 + g * 16, 16)]
                            for l in range(16):
                                v = vv[l]
                                e = q * 64 + g * 16 + l
                                eo = g * 16 + l
                                for dd in range(D // 32):
                                    w = buf[e, pl.ds(dd * 16, 16)]
                                    lo = lax.bitcast_convert_type(
                                        w << 16, jnp.float32)
                                    hi = lax.bitcast_convert_type(
                                        w & jnp.int32(-65536), jnp.float32)
                                    fbuf[eo, pl.ds(dd * 32, 16)] = lo * v
                                    fbuf[eo, pl.ds(dd * 32 + 16, 16)] = hi * v
                        pltpu.async_copy(fbuf,
                                         acc_sh.at[dst_a.at[2 * c + q]],
                                         fsem, add=True)
                return carry
            lax.fori_loop(0, half // 2, _pair, 0)
            pltpu.make_async_copy(fbuf0, acc_sh.at[dst_a.at[0]], fsem0).wait()
            pltpu.make_async_copy(fbuf1, acc_sh.at[dst_a.at[0]], fsem1).wait()
        plsc.subcore_barrier()

        for z in range((n_wb + NS - 1) // NS):
            zc = z * NS + sid

            @pl.when(zc < n_wb)
            def _():
                pltpu.sync_copy(acc_sh.at[pl.ds(zc * wb, wb)],
                                out_hbm.at[cid, pl.ds(zc * wb, wb)])

    return agg


def kernel(edge_index, adjacency_values, input_feature, W, b):
    e = edge_index.shape[1]
    grain = NW * CHUNK * 16
    e_pad = ((e + grain - 1) // grain) * grain
    nct = e_pad // (NW * CHUNK)
    pad = e_pad - e
    src = jnp.pad(edge_index[0].astype(jnp.int32), (0, pad))
    dst = jnp.pad(edge_index[1].astype(jnp.int32), (0, pad))
    vals = jnp.pad(adjacency_values, (0, pad))
    src3 = src.reshape(NW, nct, CHUNK)
    dst3 = dst.reshape(NW, 2 * nct, CHUNK // 2)
    val3 = vals.reshape(NW, nct, CHUNK)

    support = _matmul_bf16(input_feature, W[:, _PERM])
    sup_i32 = jax.lax.bitcast_convert_type(
        support.reshape(N_NODES, D // 2, 2), jnp.int32)
    partials = _make_sc_aggregate(nct)(src3, dst3, val3, sup_i32)
    return _combine(partials, b.reshape(1, D))


# speedup vs baseline: 1.1077x; 1.0227x over previous
import functools

import jax
import jax.numpy as jnp
import numpy as np
from jax import lax
from jax.experimental import pallas as pl
from jax.experimental.pallas import tpu as pltpu
from jax.experimental.pallas import tpu_sc as plsc

N_NODES = 10000
D = 128
CHUNK = 128
NC, NS = 2, 16
N0, N1 = 96, 64
NQ = 4
Q_MAX = N0 // NQ
TOTAL_CH = NS * (N0 + N1)
PAD_CH = TOTAL_CH + Q_MAX - N1 // NQ

_PERM = np.empty(D, dtype=np.int32)
for _g in range(D // 32):
    for _i in range(16):
        _PERM[_g * 32 + 2 * _i] = _g * 32 + _i
        _PERM[_g * 32 + 2 * _i + 1] = _g * 32 + 16 + _i


def _mm_body(x_ref, w_ref, o_ref):
    o_ref[...] = jnp.dot(x_ref[...], w_ref[...],
                         preferred_element_type=jnp.float32
                         ).astype(jnp.bfloat16)


def _matmul_bf16(x, w):
    m_blk = 1000
    grid = (N_NODES // m_blk,)
    return pl.pallas_call(
        _mm_body,
        grid=grid,
        in_specs=[
            pl.BlockSpec((m_blk, D), lambda i: (i, 0)),
            pl.BlockSpec((D, D), lambda i: (0, 0)),
        ],
        out_specs=pl.BlockSpec((m_blk, D), lambda i: (i, 0)),
        out_shape=jax.ShapeDtypeStruct((N_NODES, D), jnp.bfloat16),
    )(x, w)


def _comb_body(p_ref, b_ref, o_ref):
    o_ref[...] = p_ref[0] + p_ref[1] + b_ref[...]


def _combine(partials, b2d):
    m_blk = 1000
    grid = (N_NODES // m_blk,)
    return pl.pallas_call(
        _comb_body,
        grid=grid,
        in_specs=[
            pl.BlockSpec((2, m_blk, D), lambda i: (0, i, 0)),
            pl.BlockSpec((1, D), lambda i: (0, 0)),
        ],
        out_specs=pl.BlockSpec((m_blk, D), lambda i: (i, 0)),
        out_shape=jax.ShapeDtypeStruct((N_NODES, D), jnp.float32),
    )(partials, b2d)


def _make_sc_aggregate():
    wb = 40
    n_wb = N_NODES // wb
    mesh = plsc.VectorSubcoreMesh(core_axis_name="c", subcore_axis_name="s")

    @functools.partial(
        pl.kernel,
        mesh=mesh,
        compiler_params=pltpu.CompilerParams(use_tc_tiling_on_sc=False),
        out_type=jax.ShapeDtypeStruct((NC, N_NODES, D), jnp.float32),
        scratch_types=[
            pltpu.VMEM((Q_MAX, CHUNK), jnp.int32),
            pltpu.VMEM((2 * Q_MAX, CHUNK // 2), jnp.int32),
            pltpu.VMEM((Q_MAX, CHUNK), jnp.float32),
            pltpu.VMEM((CHUNK, D // 2), jnp.int32),
            pltpu.VMEM((CHUNK, D // 2), jnp.int32),
            pltpu.VMEM((CHUNK // 2, D), jnp.float32),
            pltpu.VMEM((CHUNK // 2, D), jnp.float32),
            pltpu.VMEM_SHARED((N_NODES, D), jnp.float32),
            pltpu.SemaphoreType.DMA,
            pltpu.SemaphoreType.DMA,
            pltpu.SemaphoreType.DMA,
            pltpu.SemaphoreType.DMA,
        ],
    )
    def agg(src_hbm, dst_hbm, val_hbm, sup_hbm, out_hbm,
            src_a, dst_a, val_a, bbuf0, bbuf1, fbuf0, fbuf1, acc_sh,
            gsem0, gsem1, fsem0, fsem1):
        cid = lax.axis_index("c")
        sid = lax.axis_index("s")
        bbufs = (bbuf0, bbuf1)
        gsems = (gsem0, gsem1)
        fbufs = (fbuf0, fbuf1)
        fsems = (fsem0, fsem1)
        n_c = jnp.where(cid == 0, N0, N1)
        q_c = n_c // NQ
        chunk_base = cid * NS * N0 + sid * n_c

        def _zero_row(r, carry):
            for dd in range(D // 16):
                fbuf0[r, pl.ds(dd * 16, 16)] = jnp.zeros((16,), jnp.float32)
            return carry
        lax.fori_loop(0, wb, _zero_row, 0)
        for z in range((n_wb + NS - 1) // NS):
            zc = z * NS + sid

            @pl.when(zc < n_wb)
            def _():
                pltpu.sync_copy(fbuf0.at[pl.ds(0, wb)],
                                acc_sh.at[pl.ds(zc * wb, wb)])
        plsc.subcore_barrier()

        def _quarter(h, qcarry):
            qbase = chunk_base + h * q_c
            pltpu.sync_copy(src_hbm.at[pl.ds(qbase, Q_MAX)], src_a)
            pltpu.sync_copy(dst_hbm.at[pl.ds(2 * qbase, 2 * Q_MAX)], dst_a)
            pltpu.sync_copy(val_hbm.at[pl.ds(qbase, Q_MAX)], val_a)
            pltpu.async_copy(sup_hbm.at[src_a.at[0]], bbuf0, gsem0)
            pltpu.async_copy(sup_hbm.at[src_a.at[1]], bbuf1, gsem1)

            def _pair(i, carry):
                for b in range(2):
                    c = 2 * i + b
                    o = 1 - b
                    buf, gsem = bbufs[b], gsems[b]

                    @pl.when(jnp.logical_and(c >= 1, c + 1 < q_c))
                    def _():
                        pltpu.async_copy(sup_hbm.at[src_a.at[c + 1]],
                                         bbufs[o], gsems[o])

                    pltpu.make_async_copy(sup_hbm.at[src_a.at[c]], buf,
                                          gsem).wait()

                    for q in range(2):
                        fbuf, fsem = fbufs[q], fsems[q]

                        @pl.when(c >= 1)
                        def _():
                            pltpu.make_async_copy(
                                fbuf, acc_sh.at[dst_a.at[0]], fsem).wait()

                        @plsc.parallel_loop(0, 4, unroll=2)
                        def _scale(g):
                            vv = val_a[c, pl.ds(q * 64 + g * 16, 16)]
                            for l in range(16):
                                v = vv[l]
                                e = q * 64 + g * 16 + l
                                eo = g * 16 + l
                                for dd in range(D // 32):
                                    w = buf[e, pl.ds(dd * 16, 16)]
                                    lo = lax.bitcast_convert_type(
                                        w << 16, jnp.float32)
                                    hi = lax.bitcast_convert_type(
                                        w & jnp.int32(-65536), jnp.float32)
                                    fbuf[eo, pl.ds(dd * 32, 16)] = lo * v
                                    fbuf[eo, pl.ds(dd * 32 + 16, 16)] = hi * v
                        pltpu.async_copy(fbuf,
                                         acc_sh.at[dst_a.at[2 * c + q]],
                                         fsem, add=True)
                return carry
            lax.fori_loop(0, q_c // 2, _pair, 0)
            pltpu.make_async_copy(fbuf0, acc_sh.at[dst_a.at[0]], fsem0).wait()
            pltpu.make_async_copy(fbuf1, acc_sh.at[dst_a.at[0]], fsem1).wait()
            return qcarry
        lax.fori_loop(0, NQ, _quarter, 0)
        plsc.subcore_barrier()

        for z in range((n_wb + NS - 1) // NS):
            zc = z * NS + sid

            @pl.when(zc < n_wb)
            def _():
                pltpu.sync_copy(acc_sh.at[pl.ds(zc * wb, wb)],
                                out_hbm.at[cid, pl.ds(zc * wb, wb)])

    return agg


def kernel(edge_index, adjacency_values, input_feature, W, b):
    e = edge_index.shape[1]
    e_pad = PAD_CH * CHUNK
    pad = e_pad - e
    src = jnp.pad(edge_index[0].astype(jnp.int32), (0, pad))
    dst = jnp.pad(edge_index[1].astype(jnp.int32), (0, pad))
    vals = jnp.pad(adjacency_values, (0, pad))
    src2 = src.reshape(PAD_CH, CHUNK)
    dst2 = dst.reshape(2 * PAD_CH, CHUNK // 2)
    val2 = vals.reshape(PAD_CH, CHUNK)

    support = _matmul_bf16(input_feature, W[:, _PERM])
    sup_i32 = jax.lax.bitcast_convert_type(
        support.reshape(N_NODES, D // 2, 2), jnp.int32)
    partials = _make_sc_aggregate()(src2, dst2, val2, sup_i32)
    return _combine(partials, b.reshape(1, D))
